# Initial kernel scaffold; baseline (speedup 1.0000x reference)
import jax
import jax.numpy as jnp
from jax.experimental import pallas as pl


def kernel(kpt_feature, kpt_3d, W_ge1, b_ge1, W_ge2, b_ge2, Wq, bq, Wk, bk, Wv, bv, Wga, bga, Wo, bo):
    raise NotImplementedError("write your pallas kernel here")


# speedup vs baseline: 32.2258x; 32.2258x over previous
import functools

import jax
import jax.numpy as jnp
import numpy as np
from jax.experimental import pallas as pl
from jax.experimental.pallas import tpu as pltpu
from jax.experimental.pallas import tpu_sc as plsc

KNN = 8
NSEL = KNN + 1
ROWS = 256
GW = 128


def _b16(v):
    return v.astype(jnp.bfloat16).astype(jnp.float32)


def _topk_proj_body(posT_ref, posr_ref, x_ref, wkT_ref, bk_ref, wvT_ref,
                    bv_ref, gidx_ref, nd_ref, kx_ref, ky_ref, kz_ref,
                    kf_ref, vf_ref, dist_ref):
    n = posT_ref.shape[2]
    b = pl.program_id(0)

    pall = posT_ref[0]
    xall = pall[0:1, :]
    yall = pall[1:2, :]
    zall = pall[2:3, :]
    sq_all = xall * xall + yall * yall + zall * zall

    pr = posr_ref[0]
    xr = pr[:, 0:1]
    yr = pr[:, 1:2]
    zr = pr[:, 2:3]
    sq_r = xr * xr + yr * yr + zr * zr

    xr6 = _b16(xr); yr6 = _b16(yr); zr6 = _b16(zr)
    xa6 = _b16(xall); ya6 = _b16(yall); za6 = _b16(zall)
    d2 = sq_r + sq_all - 2.0 * (xr6 * xa6 + yr6 * ya6 + zr6 * za6)
    dist = jnp.where(d2 > 1e-12, jnp.sqrt(jnp.maximum(d2, 1e-12)), 0.0)
    dist_ref[...] = dist

    iota = jax.lax.broadcasted_iota(jnp.int32, (1, n), 1)
    for t in range(NSEL):
        dcur = dist_ref[...]
        m = jnp.min(dcur, axis=1, keepdims=True)
        cand = jnp.where(dcur == m, iota, n)
        idx = jnp.min(cand, axis=1, keepdims=True)
        onehot = iota == idx
        dist_ref[...] = jnp.where(onehot, jnp.inf, dcur)
        if t >= 1:
            gidx_ref[0, :, t - 1] = idx[:, 0] + b * n
            nd_ref[0, :, t - 1] = m[:, 0]
            kx_ref[0, :, t - 1] = jnp.sum(
                jnp.where(onehot, xall, 0.0), axis=1)
            ky_ref[0, :, t - 1] = jnp.sum(
                jnp.where(onehot, yall, 0.0), axis=1)
            kz_ref[0, :, t - 1] = jnp.sum(
                jnp.where(onehot, zall, 0.0), axis=1)

    x16 = x_ref[0].astype(jnp.bfloat16)
    kf_ref[0] = jnp.dot(x16, wkT_ref[...],
                        preferred_element_type=jnp.float32) + bk_ref[...]
    vf_ref[0] = jnp.dot(x16, wvT_ref[...],
                        preferred_element_type=jnp.float32) + bv_ref[...]


def _run_topk_proj(posT, pos, x, wkT, bk2, wvT, bv2):
    b, n, d = x.shape
    grid = (b, n // ROWS)
    return pl.pallas_call(
        _topk_proj_body,
        grid=grid,
        in_specs=[
            pl.BlockSpec((1, 3, n), lambda bi, ri: (bi, 0, 0)),
            pl.BlockSpec((1, ROWS, 3), lambda bi, ri: (bi, ri, 0)),
            pl.BlockSpec((1, ROWS, d), lambda bi, ri: (bi, ri, 0)),
            pl.BlockSpec((d, d), lambda bi, ri: (0, 0)),
            pl.BlockSpec((1, d), lambda bi, ri: (0, 0)),
            pl.BlockSpec((d, d), lambda bi, ri: (0, 0)),
            pl.BlockSpec((1, d), lambda bi, ri: (0, 0)),
        ],
        out_specs=[
            pl.BlockSpec((1, ROWS, KNN), lambda bi, ri: (bi, ri, 0)),
            pl.BlockSpec((1, ROWS, KNN), lambda bi, ri: (bi, ri, 0)),
            pl.BlockSpec((1, ROWS, KNN), lambda bi, ri: (bi, ri, 0)),
            pl.BlockSpec((1, ROWS, KNN), lambda bi, ri: (bi, ri, 0)),
            pl.BlockSpec((1, ROWS, KNN), lambda bi, ri: (bi, ri, 0)),
            pl.BlockSpec((1, ROWS, d), lambda bi, ri: (bi, ri, 0)),
            pl.BlockSpec((1, ROWS, d), lambda bi, ri: (bi, ri, 0)),
        ],
        out_shape=[
            jax.ShapeDtypeStruct((b, n, KNN), jnp.int32),
            jax.ShapeDtypeStruct((b, n, KNN), jnp.float32),
            jax.ShapeDtypeStruct((b, n, KNN), jnp.float32),
            jax.ShapeDtypeStruct((b, n, KNN), jnp.float32),
            jax.ShapeDtypeStruct((b, n, KNN), jnp.float32),
            jax.ShapeDtypeStruct((b, n, d), jnp.float32),
            jax.ShapeDtypeStruct((b, n, d), jnp.float32),
        ],
        scratch_shapes=[pltpu.VMEM((ROWS, n), jnp.float32)],
        compiler_params=pltpu.CompilerParams(
            dimension_semantics=("parallel", "parallel")),
    )(posT, pos, x, wkT, bk2, wvT, bv2)


def _sc_gather_one(tab, idx_flat):
    m = idx_flat.shape[1]
    d = tab.shape[1]
    mesh = plsc.VectorSubcoreMesh(core_axis_name="core",
                                  subcore_axis_name="subcore")

    @functools.partial(
        pl.kernel,
        out_type=jax.ShapeDtypeStruct((m, d), jnp.float32),
        mesh=mesh,
    )
    def gather_kernel(tab_hbm, i_hbm, o_hbm):
        def body(i_vmem, o_vmem):
            pltpu.sync_copy(tab_hbm.at[i_vmem.at[0]], o_vmem)

        pltpu.emit_pipeline(
            body,
            grid=(m // GW,),
            in_specs=[pl.BlockSpec((1, GW), lambda i: (0, i))],
            out_specs=[pl.BlockSpec((GW, d), lambda i: (i, 0))],
            core_axis_name=("core", "subcore"),
            dimension_semantics=(pltpu.PARALLEL,),
        )(i_hbm, o_hbm)

    return gather_kernel(tab, idx_flat)


def _acos_poly(r):
    ax = jnp.abs(r)
    p = (-0.0012624911 + 0.0066700901 * ax)
    p = (-0.0170881256 + ax * p)
    p = (0.0308918810 + ax * p)
    p = (-0.0501743046 + ax * p)
    p = (0.0889789874 + ax * p)
    p = (-0.2145988016 + ax * p)
    p = (1.5707963050 + ax * p)
    base = jnp.sqrt(jnp.maximum(1.0 - ax, 0.0)) * p
    return jnp.where(r >= 0.0, base, np.pi - base)


def _cos_sin_small(y):
    y2 = y * y
    c = 1.0 + y2 * (-0.5 + y2 * (1.0 / 24.0 + y2 * (-1.0 / 720.0
        + y2 * (1.0 / 40320.0 - y2 * (1.0 / 3628800.0)))))
    s = y * (1.0 + y2 * (-1.0 / 6.0 + y2 * (1.0 / 120.0 + y2 * (-1.0 / 5040.0
        + y2 * (1.0 / 362880.0)))))
    return c, s


def _attn_geo_body(x_ref, wqT_ref, bq_ref, k3_ref, nd_ref, kx_ref, ky_ref,
                   kz_ref, kfg_ref, vfg_ref, wge1T_ref, bge1_ref, wge2T_ref,
                   bge2_ref, wgaT_ref, bga_ref, woT_ref, bo_ref, out_ref):
    d = x_ref.shape[2]
    hd = d // 4
    nh = 4

    nd = nd_ref[0]
    mean_d = jnp.mean(nd, axis=1, keepdims=True)
    dev = nd - mean_d
    std_d = jnp.sqrt(jnp.sum(dev * dev, axis=1, keepdims=True) / (KNN - 1))

    kxm = kx_ref[0]
    kym = ky_ref[0]
    kzm = kz_ref[0]
    xs = [kxm[:, k:k + 1] for k in range(KNN)]
    ys = [kym[:, k:k + 1] for k in range(KNN)]
    zs = [kzm[:, k:k + 1] for k in range(KNN)]
    cx = sum(xs) / KNN
    cy = sum(ys) / KNN
    cz = sum(zs) / KNN

    k3 = k3_ref[0]
    rcx = k3[:, 0:1] - cx
    rcy = k3[:, 1:2] - cy
    rcz = k3[:, 2:3] - cz

    a00 = a01 = a02 = a11 = a12 = a22 = 0.0
    for k in range(KNN):
        ex = _b16(xs[k] - cx)
        ey = _b16(ys[k] - cy)
        ez = _b16(zs[k] - cz)
        a00 += ex * ex
        a01 += ex * ey
        a02 += ex * ez
        a11 += ey * ey
        a12 += ey * ez
        a22 += ez * ez
    a00 /= KNN; a01 /= KNN; a02 /= KNN
    a11 /= KNN; a12 /= KNN; a22 /= KNN

    q = (a00 + a11 + a22) / 3.0
    p1 = a01 * a01 + a02 * a02 + a12 * a12
    d0 = a00 - q
    d1 = a11 - q
    d2m = a22 - q
    p2 = d0 * d0 + d1 * d1 + d2m * d2m + 2.0 * p1
    p = jnp.sqrt(jnp.maximum(p2, 0.0) / 6.0)
    ps = jnp.maximum(p, 1e-20)
    b00 = d0 / ps; b11 = d1 / ps; b22 = d2m / ps
    b01 = a01 / ps; b02 = a02 / ps; b12 = a12 / ps
    detb = (b00 * (b11 * b22 - b12 * b12)
            - b01 * (b01 * b22 - b12 * b02)
            + b02 * (b01 * b12 - b11 * b02))
    r = jnp.clip(detb / 2.0, -1.0, 1.0)
    phi = _acos_poly(r) / 3.0
    cphi, sphi = _cos_sin_small(phi)
    eig_max = q + 2.0 * p * cphi
    eig_min = q - p * (cphi + np.sqrt(3.0) * sphi)
    degen = p2 < 1e-30
    eig_max = jnp.where(degen, q, eig_max)
    eig_min = jnp.where(degen, q, eig_min)
    anis = eig_max / (eig_min + 1e-8)

    ga = jnp.concatenate([mean_d, mean_d, std_d, rcx, rcy, rcz, anis],
                         axis=1)
    h1 = jnp.maximum(
        jnp.dot(ga.astype(jnp.bfloat16), wge1T_ref[...],
                preferred_element_type=jnp.float32) + bge1_ref[...], 0.0)
    ge = jnp.dot(h1.astype(jnp.bfloat16), wge2T_ref[...],
                 preferred_element_type=jnp.float32) + bge2_ref[...]

    x16 = x_ref[0].astype(jnp.bfloat16)
    qm = jnp.dot(x16, wqT_ref[...],
                 preferred_element_type=jnp.float32) + bq_ref[...]

    kfg = kfg_ref[0]
    vfg = vfg_ref[0]
    wgaT = wgaT_ref[...]
    bga = bga_ref[...]
    scale = 1.0 / np.sqrt(hd)

    attn_cols = []
    for k in range(KNN):
        kk = kfg[:, d * k:d * (k + 1)]
        prod = qm * kk
        s_h = [jnp.sum(prod[:, hd * h:hd * (h + 1)], axis=1, keepdims=True)
               for h in range(nh)]
        s_k = jnp.concatenate(s_h, axis=1) * scale
        gf_k = jnp.concatenate(
            [nd[:, k:k + 1], mean_d, std_d, rcx, rcy, rcz, anis], axis=1)
        bias_k = jnp.dot(gf_k.astype(jnp.bfloat16), wgaT,
                         preferred_element_type=jnp.float32) + bga
        attn_cols.append(s_k + bias_k)

    agg = jnp.zeros_like(qm)
    aw_cols = [[None] * KNN for _ in range(nh)]
    for h in range(nh):
        a_h = jnp.concatenate([attn_cols[k][:, h:h + 1] for k in range(KNN)],
                              axis=1)
        a_h = a_h - jnp.max(a_h, axis=1, keepdims=True)
        e_h = jnp.exp(a_h)
        w_h = e_h / jnp.sum(e_h, axis=1, keepdims=True)
        for k in range(KNN):
            aw_cols[h][k] = w_h[:, k:k + 1]
    rr = qm.shape[0]
    for k in range(KNN):
        sc = jnp.concatenate(
            [jnp.broadcast_to(aw_cols[h][k], (rr, hd)) for h in range(nh)],
            axis=1)
        agg += sc * vfg[:, d * k:d * (k + 1)]

    out_ref[0] = ge + jnp.dot(agg.astype(jnp.bfloat16), woT_ref[...],
                              preferred_element_type=jnp.float32) + bo_ref[...]


def _run_attn_geo(x, wqT, bq2, kpt_3d, nd, kx, ky, kz, kfg, vfg,
                  wge1T, bge12, wge2T, bge22, wgaT, bga2, woT, bo2):
    b, n, d = x.shape
    grid = (b, n // ROWS)
    full = lambda bi, ri: (0, 0)
    rows = lambda bi, ri: (bi, ri, 0)
    return pl.pallas_call(
        _attn_geo_body,
        grid=grid,
        in_specs=[
            pl.BlockSpec((1, ROWS, d), rows),
            pl.BlockSpec((d, d), full),
            pl.BlockSpec((1, d), full),
            pl.BlockSpec((1, ROWS, 3), rows),
            pl.BlockSpec((1, ROWS, KNN), rows),
            pl.BlockSpec((1, ROWS, KNN), rows),
            pl.BlockSpec((1, ROWS, KNN), rows),
            pl.BlockSpec((1, ROWS, KNN), rows),
            pl.BlockSpec((1, ROWS, KNN * d), rows),
            pl.BlockSpec((1, ROWS, KNN * d), rows),
            pl.BlockSpec((7, 64), full),
            pl.BlockSpec((1, 64), full),
            pl.BlockSpec((64, d), full),
            pl.BlockSpec((1, d), full),
            pl.BlockSpec((7, 4), full),
            pl.BlockSpec((1, 4), full),
            pl.BlockSpec((d, d), full),
            pl.BlockSpec((1, d), full),
        ],
        out_specs=[pl.BlockSpec((1, ROWS, d), rows)],
        out_shape=[jax.ShapeDtypeStruct((b, n, d), jnp.float32)],
        compiler_params=pltpu.CompilerParams(
            dimension_semantics=("parallel", "parallel")),
    )(x, wqT, bq2, kpt_3d, nd, kx, ky, kz, kfg, vfg,
      wge1T, bge12, wge2T, bge22, wgaT, bga2, woT, bo2)[0]


@jax.jit
def kernel(kpt_feature, kpt_3d, W_ge1, b_ge1, W_ge2, b_ge2, Wq, bq, Wk, bk,
           Wv, bv, Wga, bga, Wo, bo):
    b, n, d = kpt_feature.shape

    posT = jnp.transpose(kpt_3d, (0, 2, 1))
    b16 = jnp.bfloat16
    gidx, nd, kx, ky, kz, kf, vf = _run_topk_proj(
        posT, kpt_3d, kpt_feature,
        Wk.T.astype(b16), bk.reshape(1, d), Wv.T.astype(b16),
        bv.reshape(1, d))

    idx_flat = gidx.reshape(1, b * n * KNN)
    kf_g = _sc_gather_one(kf.reshape(b * n, d), idx_flat)
    vf_g = _sc_gather_one(vf.reshape(b * n, d), idx_flat)

    out = _run_attn_geo(
        kpt_feature, Wq.T.astype(b16), bq.reshape(1, d), kpt_3d, nd,
        kx, ky, kz,
        kf_g.reshape(b, n, KNN * d), vf_g.reshape(b, n, KNN * d),
        W_ge1.T.astype(b16), b_ge1.reshape(1, 64),
        W_ge2.T.astype(b16), b_ge2.reshape(1, d),
        Wga.T.astype(b16), bga.reshape(1, 4),
        Wo.T.astype(b16), bo.reshape(1, d))
    return out


# speedup vs baseline: 33.7922x; 1.0486x over previous
import functools

import jax
import jax.numpy as jnp
import numpy as np
from jax.experimental import pallas as pl
from jax.experimental.pallas import tpu as pltpu
from jax.experimental.pallas import tpu_sc as plsc

KNN = 8
NSEL = KNN + 1
ROWS = 512
GW = 128


def _b16(v):
    return v.astype(jnp.bfloat16).astype(jnp.float32)


def _topk_proj_body(posT_ref, posr_ref, x_ref, wkT_ref, bk_ref, wvT_ref,
                    bv_ref, gidx_ref, nd_ref, kx_ref, ky_ref, kz_ref,
                    kf_ref, vf_ref, dist_ref):
    n = posT_ref.shape[2]
    b = pl.program_id(0)

    pall = posT_ref[0]
    xall = pall[0:1, :]
    yall = pall[1:2, :]
    zall = pall[2:3, :]
    sq_all = xall * xall + yall * yall + zall * zall

    pr = posr_ref[0]
    xr = pr[:, 0:1]
    yr = pr[:, 1:2]
    zr = pr[:, 2:3]
    sq_r = xr * xr + yr * yr + zr * zr

    xr6 = _b16(xr); yr6 = _b16(yr); zr6 = _b16(zr)
    xa6 = _b16(xall); ya6 = _b16(yall); za6 = _b16(zall)
    d2 = sq_r + sq_all - 2.0 * (xr6 * xa6 + yr6 * ya6 + zr6 * za6)
    dist = jnp.where(d2 > 1e-12, jnp.sqrt(jnp.maximum(d2, 1e-12)), 0.0)
    dist_ref[...] = dist

    iota = jax.lax.broadcasted_iota(jnp.int32, (1, n), 1)
    for t in range(NSEL):
        dcur = dist_ref[...]
        m = jnp.min(dcur, axis=1, keepdims=True)
        cand = jnp.where(dcur == m, iota, n)
        idx = jnp.min(cand, axis=1, keepdims=True)
        onehot = iota == idx
        dist_ref[...] = jnp.where(onehot, jnp.inf, dcur)
        if t >= 1:
            gidx_ref[0, :, t - 1] = idx[:, 0] + b * n
            nd_ref[0, :, t - 1] = m[:, 0]
            kx_ref[0, :, t - 1] = jnp.sum(
                jnp.where(onehot, xall, 0.0), axis=1)
            ky_ref[0, :, t - 1] = jnp.sum(
                jnp.where(onehot, yall, 0.0), axis=1)
            kz_ref[0, :, t - 1] = jnp.sum(
                jnp.where(onehot, zall, 0.0), axis=1)

    x16 = x_ref[0].astype(jnp.bfloat16)
    kf_ref[0] = jnp.dot(x16, wkT_ref[...],
                        preferred_element_type=jnp.float32) + bk_ref[...]
    vf_ref[0] = jnp.dot(x16, wvT_ref[...],
                        preferred_element_type=jnp.float32) + bv_ref[...]


def _run_topk_proj(posT, pos, x, wkT, bk2, wvT, bv2):
    b, n, d = x.shape
    grid = (b, n // ROWS)
    return pl.pallas_call(
        _topk_proj_body,
        grid=grid,
        in_specs=[
            pl.BlockSpec((1, 3, n), lambda bi, ri: (bi, 0, 0)),
            pl.BlockSpec((1, ROWS, 3), lambda bi, ri: (bi, ri, 0)),
            pl.BlockSpec((1, ROWS, d), lambda bi, ri: (bi, ri, 0)),
            pl.BlockSpec((d, d), lambda bi, ri: (0, 0)),
            pl.BlockSpec((1, d), lambda bi, ri: (0, 0)),
            pl.BlockSpec((d, d), lambda bi, ri: (0, 0)),
            pl.BlockSpec((1, d), lambda bi, ri: (0, 0)),
        ],
        out_specs=[
            pl.BlockSpec((1, ROWS, KNN), lambda bi, ri: (bi, ri, 0)),
            pl.BlockSpec((1, ROWS, KNN), lambda bi, ri: (bi, ri, 0)),
            pl.BlockSpec((1, ROWS, KNN), lambda bi, ri: (bi, ri, 0)),
            pl.BlockSpec((1, ROWS, KNN), lambda bi, ri: (bi, ri, 0)),
            pl.BlockSpec((1, ROWS, KNN), lambda bi, ri: (bi, ri, 0)),
            pl.BlockSpec((1, ROWS, d), lambda bi, ri: (bi, ri, 0)),
            pl.BlockSpec((1, ROWS, d), lambda bi, ri: (bi, ri, 0)),
        ],
        out_shape=[
            jax.ShapeDtypeStruct((b, n, KNN), jnp.int32),
            jax.ShapeDtypeStruct((b, n, KNN), jnp.float32),
            jax.ShapeDtypeStruct((b, n, KNN), jnp.float32),
            jax.ShapeDtypeStruct((b, n, KNN), jnp.float32),
            jax.ShapeDtypeStruct((b, n, KNN), jnp.float32),
            jax.ShapeDtypeStruct((b, n, d), jnp.float32),
            jax.ShapeDtypeStruct((b, n, d), jnp.float32),
        ],
        scratch_shapes=[pltpu.VMEM((ROWS, n), jnp.float32)],
        compiler_params=pltpu.CompilerParams(
            dimension_semantics=("parallel", "parallel")),
    )(posT, pos, x, wkT, bk2, wvT, bv2)


def _sc_gather_one(tab, idx_flat):
    m = idx_flat.shape[1]
    d = tab.shape[1]
    mesh = plsc.VectorSubcoreMesh(core_axis_name="core",
                                  subcore_axis_name="subcore")

    @functools.partial(
        pl.kernel,
        out_type=jax.ShapeDtypeStruct((m, d), jnp.float32),
        mesh=mesh,
    )
    def gather_kernel(tab_hbm, i_hbm, o_hbm):
        def body(i_vmem, o_vmem):
            pltpu.sync_copy(tab_hbm.at[i_vmem.at[0]], o_vmem)

        pltpu.emit_pipeline(
            body,
            grid=(m // GW,),
            in_specs=[pl.BlockSpec((1, GW), lambda i: (0, i))],
            out_specs=[pl.BlockSpec((GW, d), lambda i: (i, 0))],
            core_axis_name=("core", "subcore"),
            dimension_semantics=(pltpu.PARALLEL,),
        )(i_hbm, o_hbm)

    return gather_kernel(tab, idx_flat)


def _acos_poly(r):
    ax = jnp.abs(r)
    p = (-0.0012624911 + 0.0066700901 * ax)
    p = (-0.0170881256 + ax * p)
    p = (0.0308918810 + ax * p)
    p = (-0.0501743046 + ax * p)
    p = (0.0889789874 + ax * p)
    p = (-0.2145988016 + ax * p)
    p = (1.5707963050 + ax * p)
    base = jnp.sqrt(jnp.maximum(1.0 - ax, 0.0)) * p
    return jnp.where(r >= 0.0, base, np.pi - base)


def _cos_sin_small(y):
    y2 = y * y
    c = 1.0 + y2 * (-0.5 + y2 * (1.0 / 24.0 + y2 * (-1.0 / 720.0
        + y2 * (1.0 / 40320.0 - y2 * (1.0 / 3628800.0)))))
    s = y * (1.0 + y2 * (-1.0 / 6.0 + y2 * (1.0 / 120.0 + y2 * (-1.0 / 5040.0
        + y2 * (1.0 / 362880.0)))))
    return c, s


def _attn_geo_body(x_ref, wqT_ref, bq_ref, k3_ref, nd_ref, kx_ref, ky_ref,
                   kz_ref, kfg_ref, vfg_ref, wge1T_ref, bge1_ref, wge2T_ref,
                   bge2_ref, wgaT_ref, bga_ref, woT_ref, bo_ref, out_ref):
    d = x_ref.shape[2]
    hd = d // 4
    nh = 4

    nd = nd_ref[0]
    mean_d = jnp.mean(nd, axis=1, keepdims=True)
    dev = nd - mean_d
    std_d = jnp.sqrt(jnp.sum(dev * dev, axis=1, keepdims=True) / (KNN - 1))

    kxm = kx_ref[0]
    kym = ky_ref[0]
    kzm = kz_ref[0]
    xs = [kxm[:, k:k + 1] for k in range(KNN)]
    ys = [kym[:, k:k + 1] for k in range(KNN)]
    zs = [kzm[:, k:k + 1] for k in range(KNN)]
    cx = sum(xs) / KNN
    cy = sum(ys) / KNN
    cz = sum(zs) / KNN

    k3 = k3_ref[0]
    rcx = k3[:, 0:1] - cx
    rcy = k3[:, 1:2] - cy
    rcz = k3[:, 2:3] - cz

    a00 = a01 = a02 = a11 = a12 = a22 = 0.0
    for k in range(KNN):
        ex = _b16(xs[k] - cx)
        ey = _b16(ys[k] - cy)
        ez = _b16(zs[k] - cz)
        a00 += ex * ex
        a01 += ex * ey
        a02 += ex * ez
        a11 += ey * ey
        a12 += ey * ez
        a22 += ez * ez
    a00 /= KNN; a01 /= KNN; a02 /= KNN
    a11 /= KNN; a12 /= KNN; a22 /= KNN

    q = (a00 + a11 + a22) / 3.0
    p1 = a01 * a01 + a02 * a02 + a12 * a12
    d0 = a00 - q
    d1 = a11 - q
    d2m = a22 - q
    p2 = d0 * d0 + d1 * d1 + d2m * d2m + 2.0 * p1
    p = jnp.sqrt(jnp.maximum(p2, 0.0) / 6.0)
    ps = jnp.maximum(p, 1e-20)
    b00 = d0 / ps; b11 = d1 / ps; b22 = d2m / ps
    b01 = a01 / ps; b02 = a02 / ps; b12 = a12 / ps
    detb = (b00 * (b11 * b22 - b12 * b12)
            - b01 * (b01 * b22 - b12 * b02)
            + b02 * (b01 * b12 - b11 * b02))
    r = jnp.clip(detb / 2.0, -1.0, 1.0)
    phi = _acos_poly(r) / 3.0
    cphi, sphi = _cos_sin_small(phi)
    eig_max = q + 2.0 * p * cphi
    eig_min = q - p * (cphi + np.sqrt(3.0) * sphi)
    degen = p2 < 1e-30
    eig_max = jnp.where(degen, q, eig_max)
    eig_min = jnp.where(degen, q, eig_min)
    anis = eig_max / (eig_min + 1e-8)

    ga = jnp.concatenate([mean_d, mean_d, std_d, rcx, rcy, rcz, anis],
                         axis=1)
    h1 = jnp.maximum(
        jnp.dot(ga.astype(jnp.bfloat16), wge1T_ref[...],
                preferred_element_type=jnp.float32) + bge1_ref[...], 0.0)
    ge = jnp.dot(h1.astype(jnp.bfloat16), wge2T_ref[...],
                 preferred_element_type=jnp.float32) + bge2_ref[...]

    x16 = x_ref[0].astype(jnp.bfloat16)
    qm = jnp.dot(x16, wqT_ref[...],
                 preferred_element_type=jnp.float32) + bq_ref[...]

    kfg = kfg_ref[0]
    vfg = vfg_ref[0]
    wgaT = wgaT_ref[...]
    bga = bga_ref[...]
    scale = 1.0 / np.sqrt(hd)

    attn_cols = []
    for k in range(KNN):
        kk = kfg[:, d * k:d * (k + 1)]
        prod = qm * kk
        s_h = [jnp.sum(prod[:, hd * h:hd * (h + 1)], axis=1, keepdims=True)
               for h in range(nh)]
        s_k = jnp.concatenate(s_h, axis=1) * scale
        gf_k = jnp.concatenate(
            [nd[:, k:k + 1], mean_d, std_d, rcx, rcy, rcz, anis], axis=1)
        bias_k = jnp.dot(gf_k.astype(jnp.bfloat16), wgaT,
                         preferred_element_type=jnp.float32) + bga
        attn_cols.append(s_k + bias_k)

    agg = jnp.zeros_like(qm)
    aw_cols = [[None] * KNN for _ in range(nh)]
    for h in range(nh):
        a_h = jnp.concatenate([attn_cols[k][:, h:h + 1] for k in range(KNN)],
                              axis=1)
        a_h = a_h - jnp.max(a_h, axis=1, keepdims=True)
        e_h = jnp.exp(a_h)
        w_h = e_h / jnp.sum(e_h, axis=1, keepdims=True)
        for k in range(KNN):
            aw_cols[h][k] = w_h[:, k:k + 1]
    rr = qm.shape[0]
    for k in range(KNN):
        sc = jnp.concatenate(
            [jnp.broadcast_to(aw_cols[h][k], (rr, hd)) for h in range(nh)],
            axis=1)
        agg += sc * vfg[:, d * k:d * (k + 1)]

    out_ref[0] = ge + jnp.dot(agg.astype(jnp.bfloat16), woT_ref[...],
                              preferred_element_type=jnp.float32) + bo_ref[...]


def _run_attn_geo(x, wqT, bq2, kpt_3d, nd, kx, ky, kz, kfg, vfg,
                  wge1T, bge12, wge2T, bge22, wgaT, bga2, woT, bo2):
    b, n, d = x.shape
    grid = (b, n // ROWS)
    full = lambda bi, ri: (0, 0)
    rows = lambda bi, ri: (bi, ri, 0)
    return pl.pallas_call(
        _attn_geo_body,
        grid=grid,
        in_specs=[
            pl.BlockSpec((1, ROWS, d), rows),
            pl.BlockSpec((d, d), full),
            pl.BlockSpec((1, d), full),
            pl.BlockSpec((1, ROWS, 3), rows),
            pl.BlockSpec((1, ROWS, KNN), rows),
            pl.BlockSpec((1, ROWS, KNN), rows),
            pl.BlockSpec((1, ROWS, KNN), rows),
            pl.BlockSpec((1, ROWS, KNN), rows),
            pl.BlockSpec((1, ROWS, KNN * d), rows),
            pl.BlockSpec((1, ROWS, KNN * d), rows),
            pl.BlockSpec((7, 64), full),
            pl.BlockSpec((1, 64), full),
            pl.BlockSpec((64, d), full),
            pl.BlockSpec((1, d), full),
            pl.BlockSpec((7, 4), full),
            pl.BlockSpec((1, 4), full),
            pl.BlockSpec((d, d), full),
            pl.BlockSpec((1, d), full),
        ],
        out_specs=[pl.BlockSpec((1, ROWS, d), rows)],
        out_shape=[jax.ShapeDtypeStruct((b, n, d), jnp.float32)],
        compiler_params=pltpu.CompilerParams(
            dimension_semantics=("parallel", "parallel")),
    )(x, wqT, bq2, kpt_3d, nd, kx, ky, kz, kfg, vfg,
      wge1T, bge12, wge2T, bge22, wgaT, bga2, woT, bo2)[0]


@jax.jit
def kernel(kpt_feature, kpt_3d, W_ge1, b_ge1, W_ge2, b_ge2, Wq, bq, Wk, bk,
           Wv, bv, Wga, bga, Wo, bo):
    b, n, d = kpt_feature.shape

    posT = jnp.transpose(kpt_3d, (0, 2, 1))
    b16 = jnp.bfloat16
    gidx, nd, kx, ky, kz, kf, vf = _run_topk_proj(
        posT, kpt_3d, kpt_feature,
        Wk.T.astype(b16), bk.reshape(1, d), Wv.T.astype(b16),
        bv.reshape(1, d))

    idx_flat = gidx.reshape(1, b * n * KNN)
    kf_g = _sc_gather_one(kf.reshape(b * n, d), idx_flat)
    vf_g = _sc_gather_one(vf.reshape(b * n, d), idx_flat)

    out = _run_attn_geo(
        kpt_feature, Wq.T.astype(b16), bq.reshape(1, d), kpt_3d, nd,
        kx, ky, kz,
        kf_g.reshape(b, n, KNN * d), vf_g.reshape(b, n, KNN * d),
        W_ge1.T.astype(b16), b_ge1.reshape(1, 64),
        W_ge2.T.astype(b16), b_ge2.reshape(1, d),
        Wga.T.astype(b16), bga.reshape(1, 4),
        Wo.T.astype(b16), bo.reshape(1, d))
    return out


# speedup vs baseline: 38.1903x; 1.1302x over previous
import functools

import jax
import jax.numpy as jnp
import numpy as np
from jax.experimental import pallas as pl
from jax.experimental.pallas import tpu as pltpu
from jax.experimental.pallas import tpu_sc as plsc

KNN = 8
NSEL = KNN + 1
ROWS = 512
GW = 128


def _b16(v):
    return v.astype(jnp.bfloat16).astype(jnp.float32)


def _topk_proj_body(posT_ref, pall3_ref, posr_ref, x_ref, wkT_ref, bk_ref,
                    wvT_ref, bv_ref, gidx_ref, nd_ref, kx_ref, ky_ref,
                    kz_ref, kf_ref, vf_ref, dist_ref):
    n = posT_ref.shape[2]
    b = pl.program_id(0)

    pall = posT_ref[0]
    xall = pall[0:1, :]
    yall = pall[1:2, :]
    zall = pall[2:3, :]
    sq_all = xall * xall + yall * yall + zall * zall

    pr = posr_ref[0]
    xr = pr[:, 0:1]
    yr = pr[:, 1:2]
    zr = pr[:, 2:3]
    sq_r = xr * xr + yr * yr + zr * zr

    xr6 = _b16(xr); yr6 = _b16(yr); zr6 = _b16(zr)
    xa6 = _b16(xall); ya6 = _b16(yall); za6 = _b16(zall)
    d2 = sq_r + sq_all - 2.0 * (xr6 * xa6 + yr6 * ya6 + zr6 * za6)
    dist = jnp.where(d2 > 1e-12, jnp.sqrt(jnp.maximum(d2, 1e-12)), 0.0)
    dist_ref[...] = dist

    iota = jax.lax.broadcasted_iota(jnp.int32, (1, n), 1)
    for t in range(NSEL):
        dcur = dist_ref[...]
        m = jnp.min(dcur, axis=1, keepdims=True)
        cand = jnp.where(dcur == m, iota, n)
        idx = jnp.min(cand, axis=1, keepdims=True)
        onehot = iota == idx
        dist_ref[...] = jnp.where(onehot, jnp.inf, dcur)
        if t >= 1:
            gidx_ref[0, :, t - 1] = idx[:, 0] + b * n
            nd_ref[0, :, t - 1] = m[:, 0]
            kp = jnp.dot(onehot.astype(jnp.float32), pall3_ref[0],
                         preferred_element_type=jnp.float32)
            kx_ref[0, :, t - 1] = kp[:, 0]
            ky_ref[0, :, t - 1] = kp[:, 1]
            kz_ref[0, :, t - 1] = kp[:, 2]

    x16 = x_ref[0].astype(jnp.bfloat16)
    kf_ref[0] = jnp.dot(x16, wkT_ref[...],
                        preferred_element_type=jnp.float32) + bk_ref[...]
    vf_ref[0] = jnp.dot(x16, wvT_ref[...],
                        preferred_element_type=jnp.float32) + bv_ref[...]


def _run_topk_proj(posT, pall3, pos, x, wkT, bk2, wvT, bv2):
    b, n, d = x.shape
    grid = (b, n // ROWS)
    return pl.pallas_call(
        _topk_proj_body,
        grid=grid,
        in_specs=[
            pl.BlockSpec((1, 3, n), lambda bi, ri: (bi, 0, 0)),
            pl.BlockSpec((1, n, 3), lambda bi, ri: (bi, 0, 0)),
            pl.BlockSpec((1, ROWS, 3), lambda bi, ri: (bi, ri, 0)),
            pl.BlockSpec((1, ROWS, d), lambda bi, ri: (bi, ri, 0)),
            pl.BlockSpec((d, d), lambda bi, ri: (0, 0)),
            pl.BlockSpec((1, d), lambda bi, ri: (0, 0)),
            pl.BlockSpec((d, d), lambda bi, ri: (0, 0)),
            pl.BlockSpec((1, d), lambda bi, ri: (0, 0)),
        ],
        out_specs=[
            pl.BlockSpec((1, ROWS, KNN), lambda bi, ri: (bi, ri, 0)),
            pl.BlockSpec((1, ROWS, KNN), lambda bi, ri: (bi, ri, 0)),
            pl.BlockSpec((1, ROWS, KNN), lambda bi, ri: (bi, ri, 0)),
            pl.BlockSpec((1, ROWS, KNN), lambda bi, ri: (bi, ri, 0)),
            pl.BlockSpec((1, ROWS, KNN), lambda bi, ri: (bi, ri, 0)),
            pl.BlockSpec((1, ROWS, d), lambda bi, ri: (bi, ri, 0)),
            pl.BlockSpec((1, ROWS, d), lambda bi, ri: (bi, ri, 0)),
        ],
        out_shape=[
            jax.ShapeDtypeStruct((b, n, KNN), jnp.int32),
            jax.ShapeDtypeStruct((b, n, KNN), jnp.float32),
            jax.ShapeDtypeStruct((b, n, KNN), jnp.float32),
            jax.ShapeDtypeStruct((b, n, KNN), jnp.float32),
            jax.ShapeDtypeStruct((b, n, KNN), jnp.float32),
            jax.ShapeDtypeStruct((b, n, d), jnp.float32),
            jax.ShapeDtypeStruct((b, n, d), jnp.float32),
        ],
        scratch_shapes=[pltpu.VMEM((ROWS, n), jnp.float32)],
        compiler_params=pltpu.CompilerParams(
            dimension_semantics=("parallel", "parallel")),
    )(posT, pall3, pos, x, wkT, bk2, wvT, bv2)


def _sc_gather_one(tab, idx_flat):
    m = idx_flat.shape[1]
    d = tab.shape[1]
    mesh = plsc.VectorSubcoreMesh(core_axis_name="core",
                                  subcore_axis_name="subcore")

    @functools.partial(
        pl.kernel,
        out_type=jax.ShapeDtypeStruct((m, d), jnp.float32),
        mesh=mesh,
    )
    def gather_kernel(tab_hbm, i_hbm, o_hbm):
        def body(i_vmem, o_vmem):
            pltpu.sync_copy(tab_hbm.at[i_vmem.at[0]], o_vmem)

        pltpu.emit_pipeline(
            body,
            grid=(m // GW,),
            in_specs=[pl.BlockSpec((1, GW), lambda i: (0, i))],
            out_specs=[pl.BlockSpec((GW, d), lambda i: (i, 0))],
            core_axis_name=("core", "subcore"),
            dimension_semantics=(pltpu.PARALLEL,),
        )(i_hbm, o_hbm)

    return gather_kernel(tab, idx_flat)


def _acos_poly(r):
    ax = jnp.abs(r)
    p = (-0.0012624911 + 0.0066700901 * ax)
    p = (-0.0170881256 + ax * p)
    p = (0.0308918810 + ax * p)
    p = (-0.0501743046 + ax * p)
    p = (0.0889789874 + ax * p)
    p = (-0.2145988016 + ax * p)
    p = (1.5707963050 + ax * p)
    base = jnp.sqrt(jnp.maximum(1.0 - ax, 0.0)) * p
    return jnp.where(r >= 0.0, base, np.pi - base)


def _cos_sin_small(y):
    y2 = y * y
    c = 1.0 + y2 * (-0.5 + y2 * (1.0 / 24.0 + y2 * (-1.0 / 720.0
        + y2 * (1.0 / 40320.0 - y2 * (1.0 / 3628800.0)))))
    s = y * (1.0 + y2 * (-1.0 / 6.0 + y2 * (1.0 / 120.0 + y2 * (-1.0 / 5040.0
        + y2 * (1.0 / 362880.0)))))
    return c, s


def _attn_geo_body(x_ref, wqT_ref, bq_ref, k3_ref, nd_ref, kx_ref, ky_ref,
                   kz_ref, kfg_ref, vfg_ref, wge1T_ref, bge1_ref, wge2T_ref,
                   bge2_ref, wgaT_ref, bga_ref, woT_ref, bo_ref, out_ref):
    d = x_ref.shape[2]
    hd = d // 4
    nh = 4

    nd = nd_ref[0]
    mean_d = jnp.mean(nd, axis=1, keepdims=True)
    dev = nd - mean_d
    std_d = jnp.sqrt(jnp.sum(dev * dev, axis=1, keepdims=True) / (KNN - 1))

    kxm = kx_ref[0]
    kym = ky_ref[0]
    kzm = kz_ref[0]
    xs = [kxm[:, k:k + 1] for k in range(KNN)]
    ys = [kym[:, k:k + 1] for k in range(KNN)]
    zs = [kzm[:, k:k + 1] for k in range(KNN)]
    cx = sum(xs) / KNN
    cy = sum(ys) / KNN
    cz = sum(zs) / KNN

    k3 = k3_ref[0]
    rcx = k3[:, 0:1] - cx
    rcy = k3[:, 1:2] - cy
    rcz = k3[:, 2:3] - cz

    a00 = a01 = a02 = a11 = a12 = a22 = 0.0
    for k in range(KNN):
        ex = _b16(xs[k] - cx)
        ey = _b16(ys[k] - cy)
        ez = _b16(zs[k] - cz)
        a00 += ex * ex
        a01 += ex * ey
        a02 += ex * ez
        a11 += ey * ey
        a12 += ey * ez
        a22 += ez * ez
    a00 /= KNN; a01 /= KNN; a02 /= KNN
    a11 /= KNN; a12 /= KNN; a22 /= KNN

    q = (a00 + a11 + a22) / 3.0
    p1 = a01 * a01 + a02 * a02 + a12 * a12
    d0 = a00 - q
    d1 = a11 - q
    d2m = a22 - q
    p2 = d0 * d0 + d1 * d1 + d2m * d2m + 2.0 * p1
    p = jnp.sqrt(jnp.maximum(p2, 0.0) / 6.0)
    ps = jnp.maximum(p, 1e-20)
    b00 = d0 / ps; b11 = d1 / ps; b22 = d2m / ps
    b01 = a01 / ps; b02 = a02 / ps; b12 = a12 / ps
    detb = (b00 * (b11 * b22 - b12 * b12)
            - b01 * (b01 * b22 - b12 * b02)
            + b02 * (b01 * b12 - b11 * b02))
    r = jnp.clip(detb / 2.0, -1.0, 1.0)
    phi = _acos_poly(r) / 3.0
    cphi, sphi = _cos_sin_small(phi)
    eig_max = q + 2.0 * p * cphi
    eig_min = q - p * (cphi + np.sqrt(3.0) * sphi)
    degen = p2 < 1e-30
    eig_max = jnp.where(degen, q, eig_max)
    eig_min = jnp.where(degen, q, eig_min)
    anis = eig_max / (eig_min + 1e-8)

    ga = jnp.concatenate([mean_d, mean_d, std_d, rcx, rcy, rcz, anis],
                         axis=1)
    h1 = jnp.maximum(
        jnp.dot(ga.astype(jnp.bfloat16), wge1T_ref[...],
                preferred_element_type=jnp.float32) + bge1_ref[...], 0.0)
    ge = jnp.dot(h1.astype(jnp.bfloat16), wge2T_ref[...],
                 preferred_element_type=jnp.float32) + bge2_ref[...]

    x16 = x_ref[0].astype(jnp.bfloat16)
    qm = jnp.dot(x16, wqT_ref[...],
                 preferred_element_type=jnp.float32) + bq_ref[...]

    kfg = kfg_ref[0]
    vfg = vfg_ref[0]
    wgaT = wgaT_ref[...]
    bga = bga_ref[...]
    scale = 1.0 / np.sqrt(hd)

    attn_cols = []
    for k in range(KNN):
        kk = kfg[:, d * k:d * (k + 1)]
        prod = qm * kk
        s_h = [jnp.sum(prod[:, hd * h:hd * (h + 1)], axis=1, keepdims=True)
               for h in range(nh)]
        s_k = jnp.concatenate(s_h, axis=1) * scale
        gf_k = jnp.concatenate(
            [nd[:, k:k + 1], mean_d, std_d, rcx, rcy, rcz, anis], axis=1)
        bias_k = jnp.dot(gf_k.astype(jnp.bfloat16), wgaT,
                         preferred_element_type=jnp.float32) + bga
        attn_cols.append(s_k + bias_k)

    agg = jnp.zeros_like(qm)
    aw_cols = [[None] * KNN for _ in range(nh)]
    for h in range(nh):
        a_h = jnp.concatenate([attn_cols[k][:, h:h + 1] for k in range(KNN)],
                              axis=1)
        a_h = a_h - jnp.max(a_h, axis=1, keepdims=True)
        e_h = jnp.exp(a_h)
        w_h = e_h / jnp.sum(e_h, axis=1, keepdims=True)
        for k in range(KNN):
            aw_cols[h][k] = w_h[:, k:k + 1]
    rr = qm.shape[0]
    for k in range(KNN):
        sc = jnp.concatenate(
            [jnp.broadcast_to(aw_cols[h][k], (rr, hd)) for h in range(nh)],
            axis=1)
        agg += sc * vfg[:, d * k:d * (k + 1)]

    out_ref[0] = ge + jnp.dot(agg.astype(jnp.bfloat16), woT_ref[...],
                              preferred_element_type=jnp.float32) + bo_ref[...]


def _run_attn_geo(x, wqT, bq2, kpt_3d, nd, kx, ky, kz, kfg, vfg,
                  wge1T, bge12, wge2T, bge22, wgaT, bga2, woT, bo2):
    b, n, d = x.shape
    grid = (b, n // ROWS)
    full = lambda bi, ri: (0, 0)
    rows = lambda bi, ri: (bi, ri, 0)
    return pl.pallas_call(
        _attn_geo_body,
        grid=grid,
        in_specs=[
            pl.BlockSpec((1, ROWS, d), rows),
            pl.BlockSpec((d, d), full),
            pl.BlockSpec((1, d), full),
            pl.BlockSpec((1, ROWS, 3), rows),
            pl.BlockSpec((1, ROWS, KNN), rows),
            pl.BlockSpec((1, ROWS, KNN), rows),
            pl.BlockSpec((1, ROWS, KNN), rows),
            pl.BlockSpec((1, ROWS, KNN), rows),
            pl.BlockSpec((1, ROWS, KNN * d), rows),
            pl.BlockSpec((1, ROWS, KNN * d), rows),
            pl.BlockSpec((7, 64), full),
            pl.BlockSpec((1, 64), full),
            pl.BlockSpec((64, d), full),
            pl.BlockSpec((1, d), full),
            pl.BlockSpec((7, 4), full),
            pl.BlockSpec((1, 4), full),
            pl.BlockSpec((d, d), full),
            pl.BlockSpec((1, d), full),
        ],
        out_specs=[pl.BlockSpec((1, ROWS, d), rows)],
        out_shape=[jax.ShapeDtypeStruct((b, n, d), jnp.float32)],
        compiler_params=pltpu.CompilerParams(
            dimension_semantics=("parallel", "parallel")),
    )(x, wqT, bq2, kpt_3d, nd, kx, ky, kz, kfg, vfg,
      wge1T, bge12, wge2T, bge22, wgaT, bga2, woT, bo2)[0]


@jax.jit
def kernel(kpt_feature, kpt_3d, W_ge1, b_ge1, W_ge2, b_ge2, Wq, bq, Wk, bk,
           Wv, bv, Wga, bga, Wo, bo):
    b, n, d = kpt_feature.shape

    posT = jnp.transpose(kpt_3d, (0, 2, 1))
    b16 = jnp.bfloat16
    gidx, nd, kx, ky, kz, kf, vf = _run_topk_proj(
        posT, kpt_3d, kpt_3d, kpt_feature,
        Wk.T.astype(b16), bk.reshape(1, d), Wv.T.astype(b16),
        bv.reshape(1, d))

    idx_flat = gidx.reshape(1, b * n * KNN)
    kf_g = _sc_gather_one(kf.reshape(b * n, d), idx_flat)
    vf_g = _sc_gather_one(vf.reshape(b * n, d), idx_flat)

    out = _run_attn_geo(
        kpt_feature, Wq.T.astype(b16), bq.reshape(1, d), kpt_3d, nd,
        kx, ky, kz,
        kf_g.reshape(b, n, KNN * d), vf_g.reshape(b, n, KNN * d),
        W_ge1.T.astype(b16), b_ge1.reshape(1, 64),
        W_ge2.T.astype(b16), b_ge2.reshape(1, d),
        Wga.T.astype(b16), bga.reshape(1, 4),
        Wo.T.astype(b16), bo.reshape(1, d))
    return out
